# Initial kernel scaffold; baseline (speedup 1.0000x reference)
import jax
import jax.numpy as jnp
from jax.experimental import pallas as pl


def kernel(X, lens, table, W, b):
    raise NotImplementedError("write your pallas kernel here")


# speedup vs baseline: 1.3071x; 1.3071x over previous
import functools

import jax
import jax.numpy as jnp
from jax import lax
from jax.experimental import pallas as pl
from jax.experimental.pallas import tpu as pltpu
from jax.experimental.pallas import tpu_sc as plsc

_VOCAB = 1000000
_D = 32
_B = 4096
_L = 200

_NC = 2
_NS = 16
_LN = 16
_NW = _NC * _NS
_ROWS = _B // _NW
_NG = (_L + _LN - 1) // _LN
_EMB_ROWS = _NG * _LN


def _make_sc_kernel():
    mesh = plsc.VectorSubcoreMesh(core_axis_name="c", subcore_axis_name="s")

    @functools.partial(
        pl.kernel,
        mesh=mesh,
        compiler_params=pltpu.CompilerParams(
            needs_layout_passes=False, use_tc_tiling_on_sc=False),
        out_type=jax.ShapeDtypeStruct((_B,), jnp.float32),
        scratch_types=[
            pltpu.VMEM((104,), jnp.int32),
            pltpu.VMEM((104,), jnp.int32),
            pltpu.VMEM((_EMB_ROWS, _D), jnp.float32),
            pltpu.VMEM((_ROWS,), jnp.int32),
            pltpu.VMEM((48,), jnp.float32),
            pltpu.VMEM((_ROWS,), jnp.float32),
            pltpu.SemaphoreType.DMA,
        ],
    )
    def sc_kernel(x_hbm, lens_hbm, table_hbm, wb_hbm, out_hbm,
                  idx_a, idx_b, emb, lens_v, wb_v, scores, sem):
        wid = lax.axis_index("s") * _NC + lax.axis_index("c")
        base = wid * _ROWS

        pltpu.sync_copy(lens_hbm.at[pl.ds(base, _ROWS)], lens_v)
        pltpu.sync_copy(wb_hbm, wb_v)
        wv0 = wb_v[pl.ds(0, _LN)]
        wv1 = wb_v[pl.ds(_LN, _LN)]
        wv2 = wb_v[pl.ds(2 * _LN, _LN)]
        w_d = [wv0[d] for d in range(_LN)] + [wv1[d] for d in range(_LN)]
        b_vec = jnp.full((_LN,), wv2[0], jnp.float32)
        iota = lax.iota(jnp.int32, _LN)
        zero16 = jnp.zeros((_LN,), jnp.float32)

        def row_body(r, _):
            gr = base + r
            pltpu.sync_copy(x_hbm.at[pl.ds(gr * _L, 104)], idx_a)
            pltpu.sync_copy(x_hbm.at[pl.ds(gr * _L + 96, 104)], idx_b)
            cp_a = pltpu.async_copy(table_hbm.at[idx_a],
                                    emb.at[pl.ds(0, 104)], sem)
            cp_b = pltpu.async_copy(table_hbm.at[idx_b],
                                    emb.at[pl.ds(96, 104)], sem)
            cp_a.wait()
            cp_b.wait()

            len_vec = plsc.load_gather(lens_v, [jnp.full((_LN,), r, jnp.int32)])

            def grp_body(g, carry):
                s_acc, t_acc, wt_acc = carry
                tok = g * _LN + iota
                mask = tok < len_vec
                norm = zero16
                p = zero16
                for d in range(_D):
                    col = jnp.full((_LN,), d, jnp.int32)
                    a = plsc.load_gather(emb, [tok, col])
                    norm = norm + a * a
                    am = jnp.where(mask, a, 0.0)
                    p = p + w_d[d] * am
                w = jnp.where(mask, jnp.exp(norm), 0.0)
                return (s_acc + p, t_acc + w * p, wt_acc + w)

            s_acc, t_acc, wt_acc = lax.fori_loop(
                0, _NG, grp_body, (zero16, zero16, zero16))
            lenf_vec = len_vec.astype(jnp.float32)
            s_vec = jnp.full((_LN,), jnp.sum(s_acc), jnp.float32)
            t_vec = jnp.full((_LN,), jnp.sum(t_acc), jnp.float32)
            w_vec = jnp.full((_LN,), jnp.sum(wt_acc), jnp.float32)
            score_vec = s_vec / lenf_vec + t_vec / w_vec + b_vec
            plsc.store_scatter(scores, [jnp.full((_LN,), r, jnp.int32)],
                               score_vec, mask=iota == 0)
            return 0

        lax.fori_loop(0, _ROWS, row_body, 0)

        for g in range(_ROWS // _LN):
            sv = scores[pl.ds(g * _LN, _LN)]
            scores[pl.ds(g * _LN, _LN)] = 1.0 / (1.0 + jnp.exp(-sv))
        pltpu.sync_copy(scores, out_hbm.at[pl.ds(base, _ROWS)])

    return sc_kernel


_sc_kernel = _make_sc_kernel()


def kernel(X, lens, table, W, b):
    wb = jnp.concatenate(
        [W.reshape(-1).astype(jnp.float32),
         b.reshape(-1).astype(jnp.float32),
         jnp.zeros((48 - _D - 1,), jnp.float32)])
    prob = _sc_kernel(X.astype(jnp.int32).reshape(-1),
                      lens.astype(jnp.int32), table, wb)
    return prob.reshape(_B, 1)


# speedup vs baseline: 1.9637x; 1.5023x over previous
import functools

import jax
import jax.numpy as jnp
from jax import lax
from jax.experimental import pallas as pl
from jax.experimental.pallas import tpu as pltpu
from jax.experimental.pallas import tpu_sc as plsc

_VOCAB = 1000000
_D = 32
_B = 4096
_L = 200

_NC = 2
_NS = 16
_LN = 16
_NW = _NC * _NS
_ROWS = _B // _NW
_NG = (_L + _LN - 1) // _LN
_EMB_ROWS = _NG * _LN
_C1 = 104
_C2OFF = 96


def _make_sc_kernel():
    mesh = plsc.VectorSubcoreMesh(core_axis_name="c", subcore_axis_name="s")

    @functools.partial(
        pl.kernel,
        mesh=mesh,
        compiler_params=pltpu.CompilerParams(
            needs_layout_passes=False, use_tc_tiling_on_sc=False),
        out_type=jax.ShapeDtypeStruct((_B,), jnp.float32),
        scratch_types=[
            pltpu.VMEM((_ROWS * _L,), jnp.int32),
            pltpu.VMEM((_EMB_ROWS, _D), jnp.float32),
            pltpu.VMEM((_EMB_ROWS, _D), jnp.float32),
            pltpu.VMEM((_ROWS,), jnp.int32),
            pltpu.VMEM((48,), jnp.float32),
            pltpu.VMEM((_ROWS,), jnp.float32),
            pltpu.SemaphoreType.DMA,
            pltpu.SemaphoreType.DMA,
        ],
    )
    def sc_kernel(x_hbm, lens_hbm, table_hbm, wb_hbm, out_hbm,
                  xv, emb0, emb1, lens_v, wb_v, scores, sem0, sem1):
        wid = lax.axis_index("s") * _NC + lax.axis_index("c")
        base = wid * _ROWS

        pltpu.sync_copy(x_hbm.at[pl.ds(base * _L, _ROWS * _L)], xv)
        pltpu.sync_copy(lens_hbm.at[pl.ds(base, _ROWS)], lens_v)
        pltpu.sync_copy(wb_hbm, wb_v)
        wv0 = wb_v[pl.ds(0, _LN)]
        wv1 = wb_v[pl.ds(_LN, _LN)]
        wv2 = wb_v[pl.ds(2 * _LN, _LN)]
        w_d = [wv0[d] for d in range(_LN)] + [wv1[d] for d in range(_LN)]
        b_vec = jnp.full((_LN,), wv2[0], jnp.float32)
        iota = lax.iota(jnp.int32, _LN)
        zero16 = jnp.zeros((_LN,), jnp.float32)

        def row_len(r):
            lv = plsc.load_gather(lens_v, [jnp.full((_LN,), r, jnp.int32)])
            return lv

        def gather_descs(r, emb, sem):
            off = r * _L
            d1 = pltpu.make_async_copy(
                table_hbm.at[xv.at[pl.ds(off, _C1)]],
                emb.at[pl.ds(0, _C1)], sem)
            d2 = pltpu.make_async_copy(
                table_hbm.at[xv.at[pl.ds(off + _C2OFF, _C1)]],
                emb.at[pl.ds(_C2OFF, _C1)], sem)
            return d1, d2

        def start_row(r, emb, sem, ln):
            d1, d2 = gather_descs(r, emb, sem)
            d1.start()

            @pl.when(ln > _C2OFF)
            def _():
                d2.start()

        def wait_row(r, emb, sem, ln):
            d1, d2 = gather_descs(r, emb, sem)
            d1.wait()

            @pl.when(ln > _C2OFF)
            def _():
                d2.wait()

        def compute_row(r, emb, len_vec):
            ng = (len_vec[0] + (_LN - 1)) >> 4

            def grp_body(g, carry):
                s_acc, t_acc, wt_acc = carry
                tok = g * _LN + iota
                mask = tok < len_vec
                norm = zero16
                p = zero16
                for d in range(_D):
                    col = jnp.full((_LN,), d, jnp.int32)
                    a = plsc.load_gather(emb, [tok, col])
                    norm = norm + a * a
                    am = jnp.where(mask, a, 0.0)
                    p = p + w_d[d] * am
                w = jnp.where(mask, jnp.exp(norm), 0.0)
                return (s_acc + p, t_acc + w * p, wt_acc + w)

            s_acc, t_acc, wt_acc = lax.fori_loop(
                0, ng, grp_body, (zero16, zero16, zero16))
            lenf_vec = len_vec.astype(jnp.float32)
            s_vec = jnp.full((_LN,), jnp.sum(s_acc), jnp.float32)
            t_vec = jnp.full((_LN,), jnp.sum(t_acc), jnp.float32)
            w_vec = jnp.full((_LN,), jnp.sum(wt_acc), jnp.float32)
            score_vec = s_vec / lenf_vec + t_vec / w_vec + b_vec
            plsc.store_scatter(scores, [jnp.full((_LN,), r, jnp.int32)],
                               score_vec, mask=iota == 0)

        lv0 = row_len(0)
        start_row(0, emb0, sem0, lv0[0])

        def pair_body(p, _):
            r0 = 2 * p
            r1 = r0 + 1
            lv1 = row_len(r1)
            start_row(r1, emb1, sem1, lv1[0])
            lv0 = row_len(r0)
            wait_row(r0, emb0, sem0, lv0[0])
            compute_row(r0, emb0, lv0)

            @pl.when(p < _ROWS // 2 - 1)
            def _():
                lv2 = row_len(r0 + 2)
                start_row(r0 + 2, emb0, sem0, lv2[0])

            wait_row(r1, emb1, sem1, lv1[0])
            compute_row(r1, emb1, lv1)
            return 0

        lax.fori_loop(0, _ROWS // 2, pair_body, 0)

        for g in range(_ROWS // _LN):
            sv = scores[pl.ds(g * _LN, _LN)]
            scores[pl.ds(g * _LN, _LN)] = 1.0 / (1.0 + jnp.exp(-sv))
        pltpu.sync_copy(scores, out_hbm.at[pl.ds(base, _ROWS)])

    return sc_kernel


_sc_kernel = _make_sc_kernel()


def kernel(X, lens, table, W, b):
    wb = jnp.concatenate(
        [W.reshape(-1).astype(jnp.float32),
         b.reshape(-1).astype(jnp.float32),
         jnp.zeros((48 - _D - 1,), jnp.float32)])
    prob = _sc_kernel(X.astype(jnp.int32).reshape(-1),
                      lens.astype(jnp.int32), table, wb)
    return prob.reshape(_B, 1)


# speedup vs baseline: 1.9740x; 1.0052x over previous
import functools

import jax
import jax.numpy as jnp
from jax import lax
from jax.experimental import pallas as pl
from jax.experimental.pallas import tpu as pltpu
from jax.experimental.pallas import tpu_sc as plsc

_VOCAB = 1000000
_D = 32
_B = 4096
_L = 200

_NC = 2
_NS = 16
_LN = 16
_NW = _NC * _NS
_ROWS = _B // _NW
_NG = (_L + _LN - 1) // _LN
_EMB_ROWS = _NG * _LN
_C1 = 104
_C2OFF = 96
_DEPTH = 4


def _make_sc_kernel():
    mesh = plsc.VectorSubcoreMesh(core_axis_name="c", subcore_axis_name="s")

    @functools.partial(
        pl.kernel,
        mesh=mesh,
        compiler_params=pltpu.CompilerParams(
            needs_layout_passes=False, use_tc_tiling_on_sc=False),
        out_type=jax.ShapeDtypeStruct((_B,), jnp.float32),
        scratch_types=[
            pltpu.VMEM((_ROWS, _L), jnp.int32),
            [pltpu.VMEM((_EMB_ROWS, _D), jnp.float32)
             for _ in range(_DEPTH)],
            pltpu.VMEM((_ROWS,), jnp.int32),
            pltpu.VMEM((48,), jnp.float32),
            pltpu.VMEM((_ROWS,), jnp.float32),
            [pltpu.SemaphoreType.DMA for _ in range(_DEPTH)],
        ],
    )
    def sc_kernel(x_hbm, lens_hbm, table_hbm, wb_hbm, out_hbm,
                  xv, embs, lens_v, wb_v, scores, sems):
        wid = lax.axis_index("s") * _NC + lax.axis_index("c")
        base = wid * _ROWS

        pltpu.sync_copy(x_hbm.at[pl.ds(base, _ROWS)], xv)
        pltpu.sync_copy(lens_hbm.at[pl.ds(base, _ROWS)], lens_v)
        pltpu.sync_copy(wb_hbm, wb_v)
        wv0 = wb_v[pl.ds(0, _LN)]
        wv1 = wb_v[pl.ds(_LN, _LN)]
        wv2 = wb_v[pl.ds(2 * _LN, _LN)]
        w_d = [wv0[d] for d in range(_LN)] + [wv1[d] for d in range(_LN)]
        b_vec = jnp.full((_LN,), wv2[0], jnp.float32)
        iota = lax.iota(jnp.int32, _LN)
        zero16 = jnp.zeros((_LN,), jnp.float32)

        def row_len(r):
            return plsc.load_gather(lens_v, [jnp.full((_LN,), r, jnp.int32)])

        def gather_descs(r, emb, sem):
            d1 = pltpu.make_async_copy(
                table_hbm.at[xv.at[r, pl.ds(0, _C1)]],
                emb.at[pl.ds(0, _C1)], sem)
            d2 = pltpu.make_async_copy(
                table_hbm.at[xv.at[r, pl.ds(_C2OFF, _C1)]],
                emb.at[pl.ds(_C2OFF, _C1)], sem)
            return d1, d2

        def start_row(r, emb, sem, ln):
            d1, d2 = gather_descs(r, emb, sem)
            d1.start()

            @pl.when(ln > _C2OFF)
            def _():
                d2.start()

        def wait_row(r, emb, sem, ln):
            d1, d2 = gather_descs(r, emb, sem)
            d1.wait()

            @pl.when(ln > _C2OFF)
            def _():
                d2.wait()

        def compute_row(r, emb, len_vec):
            ng = (len_vec[0] + (_LN - 1)) >> 4

            def grp_body(g, carry):
                s_acc, t_acc, wt_acc = carry
                tok = g * _LN + iota
                mask = tok < len_vec
                norm = zero16
                p = zero16
                for d in range(_D):
                    col = jnp.full((_LN,), d, jnp.int32)
                    a = plsc.load_gather(emb, [tok, col])
                    norm = norm + a * a
                    am = jnp.where(mask, a, 0.0)
                    p = p + w_d[d] * am
                w = jnp.where(mask, jnp.exp(norm), 0.0)
                return (s_acc + p, t_acc + w * p, wt_acc + w)

            s_acc, t_acc, wt_acc = lax.fori_loop(
                0, ng, grp_body, (zero16, zero16, zero16))
            lenf_vec = len_vec.astype(jnp.float32)
            s_vec = jnp.full((_LN,), jnp.sum(s_acc), jnp.float32)
            t_vec = jnp.full((_LN,), jnp.sum(t_acc), jnp.float32)
            w_vec = jnp.full((_LN,), jnp.sum(wt_acc), jnp.float32)
            score_vec = s_vec / lenf_vec + t_vec / w_vec + b_vec
            plsc.store_scatter(scores, [jnp.full((_LN,), r, jnp.int32)],
                               score_vec, mask=iota == 0)

        for j in range(_DEPTH - 1):
            lv = row_len(j)
            start_row(j, embs[j], sems[j], lv[0])

        def quad_body(q, _):
            r0 = _DEPTH * q
            for j in range(_DEPTH):
                r = r0 + j
                lv = row_len(r)
                wait_row(r, embs[j], sems[j], lv[0])
                rn = r + _DEPTH - 1
                jn = (j + _DEPTH - 1) % _DEPTH

                @pl.when(rn < _ROWS)
                def _(rn=rn, jn=jn):
                    lvn = row_len(rn)
                    start_row(rn, embs[jn], sems[jn], lvn[0])

                compute_row(r, embs[j], lv)
            return 0

        lax.fori_loop(0, _ROWS // _DEPTH, quad_body, 0)

        for g in range(_ROWS // _LN):
            sv = scores[pl.ds(g * _LN, _LN)]
            scores[pl.ds(g * _LN, _LN)] = 1.0 / (1.0 + jnp.exp(-sv))
        pltpu.sync_copy(scores, out_hbm.at[pl.ds(base, _ROWS)])

    return sc_kernel


_sc_kernel = _make_sc_kernel()


def kernel(X, lens, table, W, b):
    wb = jnp.concatenate(
        [W.reshape(-1).astype(jnp.float32),
         b.reshape(-1).astype(jnp.float32),
         jnp.zeros((48 - _D - 1,), jnp.float32)])
    prob = _sc_kernel(X.astype(jnp.int32), lens.astype(jnp.int32),
                      table, wb)
    return prob.reshape(_B, 1)
